# baseline (speedup 1.0000x reference)
import functools
import jax
import jax.numpy as jnp
from jax import lax
from jax.experimental import pallas as pl
from jax.experimental.pallas import tpu as pltpu
from jax.experimental.pallas import tpu_sc as plsc

N = 65536
F = 64
SIZE = 16
CROW = 32
NC = 2
NS = 16
NW = NC * NS
ROWS_PER_W = N // NW
ELEMS_PER_W = ROWS_PER_W * F
CHUNK_ROWS = 256
CHUNK = CHUNK_ROWS * F
NCHUNKS = ROWS_PER_W // CHUNK_ROWS
TPAD = 24
CNT = F * CROW
BIAS = 8.4990234375


def _sc_histogram(xf, thr):
    mesh = plsc.VectorSubcoreMesh(core_axis_name="c", subcore_axis_name="s")

    @functools.partial(
        pl.kernel,
        out_type=jax.ShapeDtypeStruct((NW * CNT,), jnp.int32),
        mesh=mesh,
        scratch_types=[
            pltpu.VMEM((TPAD,), jnp.float32),
            pltpu.VMEM((CHUNK,), jnp.float32),
            pltpu.VMEM((CHUNK,), jnp.float32),
            pltpu.VMEM((CNT,), jnp.int32),
            pltpu.SemaphoreType.DMA,
            pltpu.SemaphoreType.DMA,
        ],
        compiler_params=pltpu.CompilerParams(needs_layout_passes=False),
    )
    def body(xf_hbm, thr_hbm, part_hbm, thr_v, xb0, xb1, cnt_v, sem0, sem1):
        cid = lax.axis_index("c")
        sid = lax.axis_index("s")
        wid = sid * NC + cid
        base = wid * ELEMS_PER_W

        pltpu.sync_copy(thr_hbm, thr_v)

        zeros16 = jnp.zeros((16,), jnp.int32)
        for k in range(CNT // 16):
            cnt_v[pl.ds(k * 16, 16)] = zeros16

        ones = jnp.full((16,), 1, jnp.int32)
        lane = lax.iota(jnp.int32, 16)
        fbase = [(lane + g * 16) * CROW for g in range(4)]

        bufs = (xb0, xb1)
        sems = (sem0, sem1)

        def start(c):
            return pltpu.async_copy(
                xf_hbm.at[pl.ds(base + c * CHUNK, CHUNK)], bufs[c % 2], sems[c % 2]
            )

        descs = [start(0), None]
        for c in range(NCHUNKS):
            if c + 1 < NCHUNKS:
                descs[(c + 1) % 2] = start(c + 1)
            descs[c % 2].wait()
            buf = bufs[c % 2]

            @plsc.parallel_loop(0, CHUNK_ROWS, unroll=8)
            def _row(r):
                roff = r * F
                for g in range(4):
                    xv = buf[pl.ds(roff + g * 16, 16)]
                    v = jnp.clip(xv * 7.5 + BIAS, 0.0, 16.0)
                    cand = v.astype(jnp.int32)
                    thi = plsc.load_gather(thr_v, [cand])
                    row = cand + (xv >= thi).astype(jnp.int32)
                    plsc.addupdate_scatter(cnt_v, [fbase[g] + row], ones)

        pltpu.sync_copy(cnt_v, part_hbm.at[pl.ds(wid * CNT, CNT)])

    return body(xf, thr)


def _final_fold(parts):
    def body(p_ref, o_ref):
        tot = p_ref[...].sum(axis=0)
        inb = tot[:, 1:SIZE]
        empty = (inb == 0).astype(jnp.int32).sum(axis=1)
        oc = tot[:, 0:1] + tot[:, SIZE : SIZE + 1]
        mean = oc.astype(jnp.float32) * (1.0 / N)
        flag = jnp.logical_or(jnp.any(mean > 0.1), jnp.any(empty > 1))
        o_ref[0, 0] = flag.astype(jnp.int32)

    return pl.pallas_call(
        body,
        out_shape=jax.ShapeDtypeStruct((1, 1), jnp.int32),
        out_specs=pl.BlockSpec(memory_space=pltpu.SMEM),
    )(parts)


@jax.jit
def kernel(x, grid):
    xf = x.reshape(-1)
    thr = jnp.concatenate(
        [
            grid[:, 0],
            jnp.full((1,), jnp.inf, jnp.float32),
            jnp.zeros((TPAD - SIZE - 1,), jnp.float32),
        ]
    )
    parts = _sc_histogram(xf, thr).reshape(NW, F, CROW)
    res = _final_fold(parts)
    return res[0, 0] > 0


# speedup vs baseline: 1.2634x; 1.2634x over previous
import functools
import jax
import jax.numpy as jnp
from jax import lax
from jax.experimental import pallas as pl
from jax.experimental.pallas import tpu as pltpu
from jax.experimental.pallas import tpu_sc as plsc

N = 65536
F = 64
SIZE = 16
CROW = 32
NC = 2
NS = 16
NW = NC * NS
ROWS_PER_W = N // NW
ELEMS_PER_W = ROWS_PER_W * F
CHUNK_ROWS = 256
CHUNK = CHUNK_ROWS * F
NCHUNKS = ROWS_PER_W // CHUNK_ROWS
TPAD = 24
CNT = F * CROW
BIAS = 8.4990234375


def _sc_histogram(xf, thr):
    mesh = plsc.VectorSubcoreMesh(core_axis_name="c", subcore_axis_name="s")

    @functools.partial(
        pl.kernel,
        out_type=jax.ShapeDtypeStruct((NW * CNT,), jnp.int32),
        mesh=mesh,
        scratch_types=[
            pltpu.VMEM((TPAD,), jnp.float32),
            pltpu.VMEM((CHUNK_ROWS, F), jnp.float32),
            pltpu.VMEM((CHUNK_ROWS, F), jnp.float32),
            pltpu.VMEM((CNT,), jnp.int32),
            pltpu.SemaphoreType.DMA,
            pltpu.SemaphoreType.DMA,
        ],
        compiler_params=pltpu.CompilerParams(
            needs_layout_passes=False, use_tc_tiling_on_sc=True
        ),
    )
    def body(xf_hbm, thr_hbm, part_hbm, thr_v, xb0, xb1, cnt_v, sem0, sem1):
        cid = lax.axis_index("c")
        sid = lax.axis_index("s")
        wid = sid * NC + cid
        base = wid * ROWS_PER_W

        pltpu.sync_copy(thr_hbm, thr_v)

        zeros16 = jnp.zeros((16,), jnp.int32)
        for k in range(CNT // 16):
            cnt_v[pl.ds(k * 16, 16)] = zeros16

        ones = jnp.full((16,), 1, jnp.int32)
        lane = lax.iota(jnp.int32, 16)
        fbase = [(lane + g * 16) * CROW for g in range(4)]

        bufs = (xb0, xb1)
        sems = (sem0, sem1)

        def start(c):
            return pltpu.async_copy(
                xf_hbm.at[pl.ds(base + c * CHUNK_ROWS, CHUNK_ROWS), :],
                bufs[c % 2],
                sems[c % 2],
            )

        descs = [start(0), None]
        for c in range(NCHUNKS):
            if c + 1 < NCHUNKS:
                descs[(c + 1) % 2] = start(c + 1)
            descs[c % 2].wait()
            buf = bufs[c % 2]

            @plsc.parallel_loop(0, CHUNK_ROWS, unroll=4)
            def _row(r):
                for g in range(4):
                    xv = buf[r, pl.ds(g * 16, 16)]
                    v = jnp.clip(xv * 7.5 + BIAS, 0.0, 16.0)
                    cand = v.astype(jnp.int32)
                    thi = plsc.load_gather(thr_v, [cand])
                    row = cand + (xv >= thi).astype(jnp.int32)
                    plsc.addupdate_scatter(cnt_v, [fbase[g] + row], ones)

        pltpu.sync_copy(cnt_v, part_hbm.at[pl.ds(wid * CNT, CNT)])

    return body(xf, thr)


def _final_fold(parts):
    def body(p_ref, o_ref):
        tot = p_ref[...].sum(axis=0)
        inb = tot[:, 1:SIZE]
        empty = (inb == 0).astype(jnp.int32).sum(axis=1)
        oc = tot[:, 0:1] + tot[:, SIZE : SIZE + 1]
        mean = oc.astype(jnp.float32) * (1.0 / N)
        flag = jnp.logical_or(jnp.any(mean > 0.1), jnp.any(empty > 1))
        o_ref[0, 0] = flag.astype(jnp.int32)

    return pl.pallas_call(
        body,
        out_shape=jax.ShapeDtypeStruct((1, 1), jnp.int32),
        out_specs=pl.BlockSpec(memory_space=pltpu.SMEM),
    )(parts)


@jax.jit
def kernel(x, grid):
    thr = jnp.concatenate(
        [
            grid[:, 0],
            jnp.full((1,), jnp.inf, jnp.float32),
            jnp.zeros((TPAD - SIZE - 1,), jnp.float32),
        ]
    )
    parts = _sc_histogram(x, thr).reshape(NW, F, CROW)
    res = _final_fold(parts)
    return res[0, 0] > 0
